# baseline (speedup 1.0000x reference)
import functools
import math

import jax
import jax.numpy as jnp
from jax import lax
from jax.experimental import pallas as pl
from jax.experimental.pallas import tpu as pltpu
from jax.experimental.pallas import tpu_sc as plsc

HIDDEN = 128
DEPTH = 3
MAX_NB = 10
B = 64
A = 256
E = 512
AFEAT = 82
BFEAT = 88
HEADS = 4
DK = HIDDEN // HEADS
QM = 40
RXNFP = 256

NC = 2
NS = 16
NW = NC * NS
HB = B // 2
CW = 128


NBUF = 4


def _make_sc_gather_sum(X, NMOL):
    CH = X // CW
    RK = MAX_NB * CH
    MPW = NMOL // NW
    NCHUNK = MPW * CH
    mesh = plsc.VectorSubcoreMesh(
        core_axis_name="c", subcore_axis_name="s",
        num_cores=NC, num_subcores=NS)

    @functools.partial(
        pl.kernel,
        out_type=jax.ShapeDtypeStruct((NMOL * X, HIDDEN), jnp.float32),
        mesh=mesh,
        scratch_types=[
            pltpu.VMEM((MPW * RK, CW), jnp.int32),
            [pltpu.VMEM((CW, HIDDEN), jnp.float32) for _ in range(NBUF)],
            [pltpu.SemaphoreType.DMA for _ in range(NBUF)],
            [pltpu.SemaphoreType.DMA for _ in range(NBUF)],
            [pltpu.SemaphoreType.DMA for _ in range(NBUF)],
        ],
    )
    def gather_sum(table_hbm, idx_hbm, out_hbm, idx_v, accs,
                   sem_a, sem_b, sem_s):
        wid = lax.axis_index("s") * NC + lax.axis_index("c")
        pltpu.sync_copy(idx_hbm.at[wid], idx_v)
        out_base = wid * (MPW * X)

        def idx_row(j, k):
            m, c = divmod(j, CH)
            return m * RK + k * CH + c

        first = [None] * NCHUNK
        adds = [None] * NCHUNK
        stores = [None] * NCHUNK

        for i in range(NCHUNK + 2):
            j = i - 2
            if 0 <= j < NCHUNK:
                bf = j % NBUF
                for h in adds[j]:
                    h.wait()
                stores[j] = pltpu.async_copy(
                    accs[bf], out_hbm.at[pl.ds(out_base + j * CW, CW)],
                    sem_s[bf])
            if i < NCHUNK:
                bf = i % NBUF
                if i >= NBUF:
                    stores[i - NBUF].wait()
                first[i] = pltpu.async_copy(
                    table_hbm.at[idx_v.at[idx_row(i, 0)]], accs[bf],
                    sem_a[bf])
            j = i - 1
            if 0 <= j < NCHUNK:
                bf = j % NBUF
                first[j].wait()
                adds[j] = [
                    pltpu.async_copy(
                        table_hbm.at[idx_v.at[idx_row(j, k)]], accs[bf],
                        sem_b[bf], add=True)
                    for k in range(1, MAX_NB)
                ]
        for j in range(max(0, NCHUNK - NBUF), NCHUNK):
            stores[j].wait()

    return gather_sum


_sc_gather_bond = _make_sc_gather_sum(E, HB)
_sc_gather_atom = _make_sc_gather_sum(A, HB)


def _bond_input(fbonds_flat, Wi, bi_row):
    TM = 2048

    def body(x_ref, w_ref, b_ref, bin_ref, msg_ref):
        acc = jnp.dot(x_ref[...], w_ref[...],
                      preferred_element_type=jnp.float32) + b_ref[...]
        bin_ref[...] = acc
        msg_ref[...] = jnp.maximum(acc, 0.0)

    n = fbonds_flat.shape[0]
    return pl.pallas_call(
        body,
        grid=(n // TM,),
        in_specs=[
            pl.BlockSpec((TM, BFEAT), lambda i: (i, 0)),
            pl.BlockSpec((BFEAT, HIDDEN), lambda i: (0, 0)),
            pl.BlockSpec((1, HIDDEN), lambda i: (0, 0)),
        ],
        out_specs=[
            pl.BlockSpec((TM, HIDDEN), lambda i: (i, 0)),
            pl.BlockSpec((TM, HIDDEN), lambda i: (i, 0)),
        ],
        out_shape=[
            jax.ShapeDtypeStruct((n, HIDDEN), jnp.float32),
            jax.ShapeDtypeStruct((n, HIDDEN), jnp.float32),
        ],
    )(fbonds_flat, Wi, bi_row)


def _message_update(g_flat, binput, Wh):
    TM = 2048

    def body(g_ref, bin_ref, w_ref, out_ref):
        out_ref[...] = jnp.maximum(
            bin_ref[...] + jnp.dot(g_ref[...], w_ref[...],
                                   preferred_element_type=jnp.float32), 0.0)

    n = g_flat.shape[0]
    return pl.pallas_call(
        body,
        grid=(n // TM,),
        in_specs=[
            pl.BlockSpec((TM, HIDDEN), lambda i: (i, 0)),
            pl.BlockSpec((TM, HIDDEN), lambda i: (i, 0)),
            pl.BlockSpec((HIDDEN, HIDDEN), lambda i: (0, 0)),
        ],
        out_specs=pl.BlockSpec((TM, HIDDEN), lambda i: (i, 0)),
        out_shape=jax.ShapeDtypeStruct((n, HIDDEN), jnp.float32),
    )(g_flat, binput, Wh)


def _atom_head(fa_flat, ga_flat, qm_flat, mask_tiles, Wo_a, Wo_h, bo_row,
               W0_h, W0_q):
    MPB = 8
    TM = MPB * A

    def body(fa_ref, ga_ref, qm_ref, m_ref, woa_ref, woh_ref, bo_ref,
             w0h_ref, w0q_ref, out_ref):
        h1 = jnp.maximum(
            jnp.dot(fa_ref[...], woa_ref[...],
                    preferred_element_type=jnp.float32)
            + jnp.dot(ga_ref[...], woh_ref[...],
                      preferred_element_type=jnp.float32)
            + bo_ref[...], 0.0)
        r = (jnp.dot(h1, w0h_ref[...], preferred_element_type=jnp.float32)
             + jnp.dot(qm_ref[...], w0q_ref[...],
                       preferred_element_type=jnp.float32))
        sel = (lax.broadcasted_iota(jnp.int32, (MPB, TM), 1) // A
               == lax.broadcasted_iota(jnp.int32, (MPB, TM), 0)
               ).astype(jnp.float32) * m_ref[0]
        out_ref[...] = jnp.dot(sel, r, preferred_element_type=jnp.float32)

    nmol = fa_flat.shape[0] // A
    return pl.pallas_call(
        body,
        grid=(nmol // MPB,),
        in_specs=[
            pl.BlockSpec((TM, AFEAT), lambda i: (i, 0)),
            pl.BlockSpec((TM, HIDDEN), lambda i: (i, 0)),
            pl.BlockSpec((TM, QM), lambda i: (i, 0)),
            pl.BlockSpec((1, 1, TM), lambda i: (i, 0, 0)),
            pl.BlockSpec((AFEAT, HIDDEN), lambda i: (0, 0)),
            pl.BlockSpec((HIDDEN, HIDDEN), lambda i: (0, 0)),
            pl.BlockSpec((1, HIDDEN), lambda i: (0, 0)),
            pl.BlockSpec((HIDDEN, HIDDEN), lambda i: (0, 0)),
            pl.BlockSpec((QM, HIDDEN), lambda i: (0, 0)),
        ],
        out_specs=pl.BlockSpec((MPB, HIDDEN), lambda i: (i, 0)),
        out_shape=jax.ShapeDtypeStruct((nmol, HIDDEN), jnp.float32),
    )(fa_flat, ga_flat, qm_flat, mask_tiles, Wo_a, Wo_h, bo_row, W0_h, W0_q)


def _attention_head(mol, rxnfp, Wq, bq_row, Wk, bk_row, Wv, bv_row,
                    Wout, bout_row, Wrs_top, Wrs_bot, brs_row):
    def body(m_ref, r_ref, wq_ref, bq_ref, wk_ref, bk_ref, wv_ref, bv_ref,
             wo_ref, bo_ref, wt_ref, wb_ref, br_ref, out_ref):
        m = m_ref[...]
        r = r_ref[...]
        q = jnp.dot(m, wq_ref[...], preferred_element_type=jnp.float32) + bq_ref[...]
        k = jnp.dot(r, wk_ref[...], preferred_element_type=jnp.float32) + bk_ref[...]
        v = jnp.dot(r, wv_ref[...], preferred_element_type=jnp.float32) + bv_ref[...]
        g = (lax.broadcasted_iota(jnp.int32, (HIDDEN, HEADS), 0) // DK
             == lax.broadcasted_iota(jnp.int32, (HIDDEN, HEADS), 1)
             ).astype(jnp.float32)
        gt = (lax.broadcasted_iota(jnp.int32, (HEADS, HIDDEN), 1) // DK
              == lax.broadcasted_iota(jnp.int32, (HEADS, HIDDEN), 0)
              ).astype(jnp.float32)
        s = jnp.dot(q * k, g, preferred_element_type=jnp.float32) \
            * (1.0 / math.sqrt(DK))
        s = s - jnp.max(s, axis=1, keepdims=True)
        e = jnp.exp(s)
        w = e / jnp.sum(e, axis=1, keepdims=True)
        wb = jnp.dot(w, gt, preferred_element_type=jnp.float32)
        att = jnp.dot(wb * v, wo_ref[...],
                      preferred_element_type=jnp.float32) + bo_ref[...]
        out_ref[...] = (
            jnp.dot(m, wt_ref[...], preferred_element_type=jnp.float32)
            + jnp.dot(att, wb_ref[...], preferred_element_type=jnp.float32)
            + br_ref[...])

    return pl.pallas_call(
        body,
        out_shape=jax.ShapeDtypeStruct((B, 1), jnp.float32),
    )(mol, rxnfp, Wq, bq_row, Wk, bk_row, Wv, bv_row, Wout, bout_row,
      Wrs_top, Wrs_bot, brs_row)


def _flat_indices(graph, X):
    nmol = graph.shape[0]
    offs = (jnp.arange(nmol, dtype=jnp.int32) * E)[:, None, None]
    fi = (graph.astype(jnp.int32) + offs)
    fi = jnp.transpose(fi, (0, 2, 1))
    return fi.reshape(NW, (nmol // NW) * MAX_NB * (X // CW), CW)


def kernel(fatoms, fbonds, atom_graph, bond_graph, num_nbs, atom_mask,
           bond_mask, core_mask, fatom_qm, res_rxnfp, Wi, bi, Wh, Wo, bo,
           W_rs0, Wq, bq, Wk, bk, Wv, bv, Wout, bout, W_rs, b_rs):
    bi_row = bi.reshape(1, HIDDEN)
    bo_row = bo.reshape(1, HIDDEN)

    mols = []
    for h in range(2):
        sl = slice(h * HB, (h + 1) * HB)
        binput, msg = _bond_input(
            fbonds[sl].reshape(HB * E, BFEAT), Wi, bi_row)
        fi_bond = _flat_indices(bond_graph[sl], E)
        for _ in range(DEPTH - 1):
            g = _sc_gather_bond(msg, fi_bond)
            msg = _message_update(g, binput, Wh)
        ga = _sc_gather_atom(msg, _flat_indices(atom_graph[sl], A))
        mask_tiles = (atom_mask[sl] * core_mask[sl]).reshape(
            HB // 8, 1, 8 * A)
        mols.append(_atom_head(
            fatoms[sl].reshape(HB * A, AFEAT), ga,
            fatom_qm[sl].reshape(HB * A, QM), mask_tiles,
            Wo[:AFEAT], Wo[AFEAT:], bo_row,
            W_rs0[:HIDDEN], W_rs0[HIDDEN:]))
    mol = jnp.concatenate(mols, axis=0)

    return _attention_head(
        mol, res_rxnfp, Wq, bq.reshape(1, HIDDEN), Wk, bk.reshape(1, HIDDEN),
        Wv, bv.reshape(1, HIDDEN), Wout, bout.reshape(1, HIDDEN),
        W_rs[:HIDDEN], W_rs[HIDDEN:], b_rs.reshape(1, 1))


# speedup vs baseline: 1.0424x; 1.0424x over previous
import functools
import math

import jax
import jax.numpy as jnp
from jax import lax
from jax.experimental import pallas as pl
from jax.experimental.pallas import tpu as pltpu
from jax.experimental.pallas import tpu_sc as plsc

HIDDEN = 128
DEPTH = 3
MAX_NB = 10
B = 64
A = 256
E = 512
AFEAT = 82
BFEAT = 88
HEADS = 4
DK = HIDDEN // HEADS
QM = 40
RXNFP = 256

NC = 2
NS = 16
NW = NC * NS
HB = B // 2
CW = 128


NBUF = 4


def _make_sc_gather_sum(X, NMOL):
    CH = X // CW
    RK = MAX_NB * CH
    MPW = NMOL // NW
    NCHUNK = MPW * CH
    mesh = plsc.VectorSubcoreMesh(
        core_axis_name="c", subcore_axis_name="s",
        num_cores=NC, num_subcores=NS)

    @functools.partial(
        pl.kernel,
        out_type=jax.ShapeDtypeStruct((NMOL * X, HIDDEN), jnp.float32),
        mesh=mesh,
        scratch_types=[
            pltpu.VMEM((MPW * RK, CW), jnp.int32),
            [pltpu.VMEM((CW, HIDDEN), jnp.float32) for _ in range(NBUF)],
            [pltpu.SemaphoreType.DMA for _ in range(NBUF)],
            [pltpu.SemaphoreType.DMA for _ in range(NBUF)],
            [pltpu.SemaphoreType.DMA for _ in range(NBUF)],
        ],
    )
    def gather_sum(table_hbm, idx_hbm, out_hbm, idx_v, accs,
                   sem_a, sem_b, sem_s):
        wid = lax.axis_index("s") * NC + lax.axis_index("c")
        pltpu.sync_copy(idx_hbm.at[wid], idx_v)
        out_base = wid * (MPW * X)

        def idx_row(j, k):
            m, c = divmod(j, CH)
            return m * RK + k * CH + c

        first = [None] * NCHUNK
        adds = [None] * NCHUNK
        stores = [None] * NCHUNK

        for i in range(NCHUNK + 2):
            j = i - 2
            if 0 <= j < NCHUNK:
                bf = j % NBUF
                for h in adds[j]:
                    h.wait()
                stores[j] = pltpu.async_copy(
                    accs[bf], out_hbm.at[pl.ds(out_base + j * CW, CW)],
                    sem_s[bf])
            if i < NCHUNK:
                bf = i % NBUF
                if i >= NBUF:
                    stores[i - NBUF].wait()
                first[i] = pltpu.async_copy(
                    table_hbm.at[idx_v.at[idx_row(i, 0)]], accs[bf],
                    sem_a[bf])
            j = i - 1
            if 0 <= j < NCHUNK:
                bf = j % NBUF
                first[j].wait()
                adds[j] = [
                    pltpu.async_copy(
                        table_hbm.at[idx_v.at[idx_row(j, k)]], accs[bf],
                        sem_b[bf], add=True)
                    for k in range(1, MAX_NB)
                ]
        for j in range(max(0, NCHUNK - NBUF), NCHUNK):
            stores[j].wait()

    return gather_sum


_sc_gather_bond = _make_sc_gather_sum(E, B)
_sc_gather_atom = _make_sc_gather_sum(A, B)


def _bond_input(fbonds_flat, Wi, bi_row):
    TM = 4096

    def body(x_ref, w_ref, b_ref, bin_ref, msg_ref):
        acc = jnp.dot(x_ref[...], w_ref[...],
                      preferred_element_type=jnp.float32) + b_ref[...]
        bin_ref[...] = acc
        msg_ref[...] = jnp.maximum(acc, 0.0)

    n = fbonds_flat.shape[0]
    return pl.pallas_call(
        body,
        grid=(n // TM,),
        in_specs=[
            pl.BlockSpec((TM, BFEAT), lambda i: (i, 0)),
            pl.BlockSpec((BFEAT, HIDDEN), lambda i: (0, 0)),
            pl.BlockSpec((1, HIDDEN), lambda i: (0, 0)),
        ],
        out_specs=[
            pl.BlockSpec((TM, HIDDEN), lambda i: (i, 0)),
            pl.BlockSpec((TM, HIDDEN), lambda i: (i, 0)),
        ],
        out_shape=[
            jax.ShapeDtypeStruct((n, HIDDEN), jnp.float32),
            jax.ShapeDtypeStruct((n, HIDDEN), jnp.float32),
        ],
    )(fbonds_flat, Wi, bi_row)


def _message_update(g_flat, binput, Wh):
    TM = 4096

    def body(g_ref, bin_ref, w_ref, out_ref):
        out_ref[...] = jnp.maximum(
            bin_ref[...] + jnp.dot(g_ref[...], w_ref[...],
                                   preferred_element_type=jnp.float32), 0.0)

    n = g_flat.shape[0]
    return pl.pallas_call(
        body,
        grid=(n // TM,),
        in_specs=[
            pl.BlockSpec((TM, HIDDEN), lambda i: (i, 0)),
            pl.BlockSpec((TM, HIDDEN), lambda i: (i, 0)),
            pl.BlockSpec((HIDDEN, HIDDEN), lambda i: (0, 0)),
        ],
        out_specs=pl.BlockSpec((TM, HIDDEN), lambda i: (i, 0)),
        out_shape=jax.ShapeDtypeStruct((n, HIDDEN), jnp.float32),
    )(g_flat, binput, Wh)


def _atom_head(fa_flat, ga_flat, qm_flat, mask_tiles, Wo_a, Wo_h, bo_row,
               W0_h, W0_q):
    MPB = 8
    TM = MPB * A

    def body(fa_ref, ga_ref, qm_ref, m_ref, woa_ref, woh_ref, bo_ref,
             w0h_ref, w0q_ref, out_ref):
        h1 = jnp.maximum(
            jnp.dot(fa_ref[...], woa_ref[...],
                    preferred_element_type=jnp.float32)
            + jnp.dot(ga_ref[...], woh_ref[...],
                      preferred_element_type=jnp.float32)
            + bo_ref[...], 0.0)
        r = (jnp.dot(h1, w0h_ref[...], preferred_element_type=jnp.float32)
             + jnp.dot(qm_ref[...], w0q_ref[...],
                       preferred_element_type=jnp.float32))
        sel = (lax.broadcasted_iota(jnp.int32, (MPB, TM), 1) // A
               == lax.broadcasted_iota(jnp.int32, (MPB, TM), 0)
               ).astype(jnp.float32) * m_ref[0]
        out_ref[...] = jnp.dot(sel, r, preferred_element_type=jnp.float32)

    nmol = fa_flat.shape[0] // A
    return pl.pallas_call(
        body,
        grid=(nmol // MPB,),
        in_specs=[
            pl.BlockSpec((TM, AFEAT), lambda i: (i, 0)),
            pl.BlockSpec((TM, HIDDEN), lambda i: (i, 0)),
            pl.BlockSpec((TM, QM), lambda i: (i, 0)),
            pl.BlockSpec((1, 1, TM), lambda i: (i, 0, 0)),
            pl.BlockSpec((AFEAT, HIDDEN), lambda i: (0, 0)),
            pl.BlockSpec((HIDDEN, HIDDEN), lambda i: (0, 0)),
            pl.BlockSpec((1, HIDDEN), lambda i: (0, 0)),
            pl.BlockSpec((HIDDEN, HIDDEN), lambda i: (0, 0)),
            pl.BlockSpec((QM, HIDDEN), lambda i: (0, 0)),
        ],
        out_specs=pl.BlockSpec((MPB, HIDDEN), lambda i: (i, 0)),
        out_shape=jax.ShapeDtypeStruct((nmol, HIDDEN), jnp.float32),
    )(fa_flat, ga_flat, qm_flat, mask_tiles, Wo_a, Wo_h, bo_row, W0_h, W0_q)


def _attention_head(mol, rxnfp, Wq, bq_row, Wk, bk_row, Wv, bv_row,
                    Wout, bout_row, Wrs_top, Wrs_bot, brs_row):
    def body(m_ref, r_ref, wq_ref, bq_ref, wk_ref, bk_ref, wv_ref, bv_ref,
             wo_ref, bo_ref, wt_ref, wb_ref, br_ref, out_ref):
        m = m_ref[...]
        r = r_ref[...]
        q = jnp.dot(m, wq_ref[...], preferred_element_type=jnp.float32) + bq_ref[...]
        k = jnp.dot(r, wk_ref[...], preferred_element_type=jnp.float32) + bk_ref[...]
        v = jnp.dot(r, wv_ref[...], preferred_element_type=jnp.float32) + bv_ref[...]
        g = (lax.broadcasted_iota(jnp.int32, (HIDDEN, HEADS), 0) // DK
             == lax.broadcasted_iota(jnp.int32, (HIDDEN, HEADS), 1)
             ).astype(jnp.float32)
        gt = (lax.broadcasted_iota(jnp.int32, (HEADS, HIDDEN), 1) // DK
              == lax.broadcasted_iota(jnp.int32, (HEADS, HIDDEN), 0)
              ).astype(jnp.float32)
        s = jnp.dot(q * k, g, preferred_element_type=jnp.float32) \
            * (1.0 / math.sqrt(DK))
        s = s - jnp.max(s, axis=1, keepdims=True)
        e = jnp.exp(s)
        w = e / jnp.sum(e, axis=1, keepdims=True)
        wb = jnp.dot(w, gt, preferred_element_type=jnp.float32)
        att = jnp.dot(wb * v, wo_ref[...],
                      preferred_element_type=jnp.float32) + bo_ref[...]
        out_ref[...] = (
            jnp.dot(m, wt_ref[...], preferred_element_type=jnp.float32)
            + jnp.dot(att, wb_ref[...], preferred_element_type=jnp.float32)
            + br_ref[...])

    return pl.pallas_call(
        body,
        out_shape=jax.ShapeDtypeStruct((B, 1), jnp.float32),
    )(mol, rxnfp, Wq, bq_row, Wk, bk_row, Wv, bv_row, Wout, bout_row,
      Wrs_top, Wrs_bot, brs_row)


def _flat_indices(graph, X):
    nmol = graph.shape[0]
    offs = (jnp.arange(nmol, dtype=jnp.int32) * E)[:, None, None]
    fi = (graph.astype(jnp.int32) + offs)
    fi = jnp.transpose(fi, (0, 2, 1))
    return fi.reshape(NW, (nmol // NW) * MAX_NB * (X // CW), CW)


def kernel(fatoms, fbonds, atom_graph, bond_graph, num_nbs, atom_mask,
           bond_mask, core_mask, fatom_qm, res_rxnfp, Wi, bi, Wh, Wo, bo,
           W_rs0, Wq, bq, Wk, bk, Wv, bv, Wout, bout, W_rs, b_rs):
    bi_row = bi.reshape(1, HIDDEN)
    bo_row = bo.reshape(1, HIDDEN)

    binput, msg = _bond_input(fbonds.reshape(B * E, BFEAT), Wi, bi_row)
    fi_bond = _flat_indices(bond_graph, E)
    fi_atom = _flat_indices(atom_graph, A)

    for _ in range(DEPTH - 1):
        g = _sc_gather_bond(msg, fi_bond)
        msg = _message_update(g, binput, Wh)
    ga = _sc_gather_atom(msg, fi_atom)

    mask_tiles = (atom_mask * core_mask).reshape(B // 8, 1, 8 * A)
    mol = _atom_head(
        fatoms.reshape(B * A, AFEAT), ga, fatom_qm.reshape(B * A, QM),
        mask_tiles, Wo[:AFEAT], Wo[AFEAT:], bo_row,
        W_rs0[:HIDDEN], W_rs0[HIDDEN:])

    return _attention_head(
        mol, res_rxnfp, Wq, bq.reshape(1, HIDDEN), Wk, bk.reshape(1, HIDDEN),
        Wv, bv.reshape(1, HIDDEN), Wout, bout.reshape(1, HIDDEN),
        W_rs[:HIDDEN], W_rs[HIDDEN:], b_rs.reshape(1, 1))


# speedup vs baseline: 1.0588x; 1.0157x over previous
import functools
import math

import jax
import jax.numpy as jnp
from jax import lax
from jax.experimental import pallas as pl
from jax.experimental.pallas import tpu as pltpu
from jax.experimental.pallas import tpu_sc as plsc

HIDDEN = 128
DEPTH = 3
MAX_NB = 10
B = 64
A = 256
E = 512
AFEAT = 82
BFEAT = 88
HEADS = 4
DK = HIDDEN // HEADS
QM = 40
RXNFP = 256

NC = 2
NS = 16
NW = NC * NS
HB = B // 2
CW = 128


NBUF = 4


def _make_sc_gather_sum(X, NMOL):
    CH = X // CW
    RK = MAX_NB * CH
    MPW = NMOL // NW
    NCHUNK = MPW * CH
    mesh = plsc.VectorSubcoreMesh(
        core_axis_name="c", subcore_axis_name="s",
        num_cores=NC, num_subcores=NS)

    @functools.partial(
        pl.kernel,
        out_type=jax.ShapeDtypeStruct((NMOL * X, HIDDEN), jnp.float32),
        mesh=mesh,
        scratch_types=[
            pltpu.VMEM((MPW * RK, CW), jnp.int32),
            [pltpu.VMEM((CW, HIDDEN), jnp.float32) for _ in range(NBUF)],
            [pltpu.SemaphoreType.DMA for _ in range(NBUF)],
            [pltpu.SemaphoreType.DMA for _ in range(NBUF)],
            [pltpu.SemaphoreType.DMA for _ in range(NBUF)],
        ],
    )
    def gather_sum(table_hbm, idx_hbm, out_hbm, idx_v, accs,
                   sem_a, sem_b, sem_s):
        wid = lax.axis_index("s") * NC + lax.axis_index("c")
        pltpu.sync_copy(idx_hbm.at[wid], idx_v)
        out_base = wid * (MPW * X)

        def idx_row(j, k):
            m, c = divmod(j, CH)
            return m * RK + k * CH + c

        first = [None] * NCHUNK
        adds = [None] * NCHUNK
        stores = [None] * NCHUNK

        for i in range(NCHUNK + 2):
            j = i - 2
            if 0 <= j < NCHUNK:
                bf = j % NBUF
                for h in adds[j]:
                    h.wait()
                stores[j] = pltpu.async_copy(
                    accs[bf], out_hbm.at[pl.ds(out_base + j * CW, CW)],
                    sem_s[bf])
            if i < NCHUNK:
                bf = i % NBUF
                if i >= NBUF:
                    stores[i - NBUF].wait()
                first[i] = pltpu.async_copy(
                    table_hbm.at[idx_v.at[idx_row(i, 0)]], accs[bf],
                    sem_a[bf])
            j = i - 1
            if 0 <= j < NCHUNK:
                bf = j % NBUF
                first[j].wait()
                adds[j] = [
                    pltpu.async_copy(
                        table_hbm.at[idx_v.at[idx_row(j, k)]], accs[bf],
                        sem_b[bf], add=True)
                    for k in range(1, MAX_NB)
                ]
        for j in range(max(0, NCHUNK - NBUF), NCHUNK):
            stores[j].wait()

    return gather_sum


_sc_gather_bond = _make_sc_gather_sum(E, B)
_sc_gather_atom = _make_sc_gather_sum(A, B)


def _bond_input(fbonds_flat, Wi, bi_row):
    TM = 8192

    def body(x_ref, w_ref, b_ref, bin_ref, msg_ref):
        acc = jnp.dot(x_ref[...], w_ref[...],
                      preferred_element_type=jnp.float32) + b_ref[...]
        bin_ref[...] = acc
        msg_ref[...] = jnp.maximum(acc, 0.0)

    n = fbonds_flat.shape[0]
    return pl.pallas_call(
        body,
        grid=(n // TM,),
        in_specs=[
            pl.BlockSpec((TM, BFEAT), lambda i: (i, 0)),
            pl.BlockSpec((BFEAT, HIDDEN), lambda i: (0, 0)),
            pl.BlockSpec((1, HIDDEN), lambda i: (0, 0)),
        ],
        out_specs=[
            pl.BlockSpec((TM, HIDDEN), lambda i: (i, 0)),
            pl.BlockSpec((TM, HIDDEN), lambda i: (i, 0)),
        ],
        out_shape=[
            jax.ShapeDtypeStruct((n, HIDDEN), jnp.float32),
            jax.ShapeDtypeStruct((n, HIDDEN), jnp.float32),
        ],
    )(fbonds_flat, Wi, bi_row)


def _message_update(g_flat, binput, Wh):
    TM = 8192

    def body(g_ref, bin_ref, w_ref, out_ref):
        out_ref[...] = jnp.maximum(
            bin_ref[...] + jnp.dot(g_ref[...], w_ref[...],
                                   preferred_element_type=jnp.float32), 0.0)

    n = g_flat.shape[0]
    return pl.pallas_call(
        body,
        grid=(n // TM,),
        in_specs=[
            pl.BlockSpec((TM, HIDDEN), lambda i: (i, 0)),
            pl.BlockSpec((TM, HIDDEN), lambda i: (i, 0)),
            pl.BlockSpec((HIDDEN, HIDDEN), lambda i: (0, 0)),
        ],
        out_specs=pl.BlockSpec((TM, HIDDEN), lambda i: (i, 0)),
        out_shape=jax.ShapeDtypeStruct((n, HIDDEN), jnp.float32),
    )(g_flat, binput, Wh)


def _atom_head(fa_flat, ga_flat, qm_flat, mask_tiles, Wo_a, Wo_h, bo_row,
               W0_h, W0_q):
    MPB = 8
    TM = MPB * A

    def body(fa_ref, ga_ref, qm_ref, m_ref, woa_ref, woh_ref, bo_ref,
             w0h_ref, w0q_ref, out_ref):
        h1 = jnp.maximum(
            jnp.dot(fa_ref[...], woa_ref[...],
                    preferred_element_type=jnp.float32)
            + jnp.dot(ga_ref[...], woh_ref[...],
                      preferred_element_type=jnp.float32)
            + bo_ref[...], 0.0)
        r = (jnp.dot(h1, w0h_ref[...], preferred_element_type=jnp.float32)
             + jnp.dot(qm_ref[...], w0q_ref[...],
                       preferred_element_type=jnp.float32))
        sel = (lax.broadcasted_iota(jnp.int32, (MPB, TM), 1) // A
               == lax.broadcasted_iota(jnp.int32, (MPB, TM), 0)
               ).astype(jnp.float32) * m_ref[0]
        out_ref[...] = jnp.dot(sel, r, preferred_element_type=jnp.float32)

    nmol = fa_flat.shape[0] // A
    return pl.pallas_call(
        body,
        grid=(nmol // MPB,),
        in_specs=[
            pl.BlockSpec((TM, AFEAT), lambda i: (i, 0)),
            pl.BlockSpec((TM, HIDDEN), lambda i: (i, 0)),
            pl.BlockSpec((TM, QM), lambda i: (i, 0)),
            pl.BlockSpec((1, 1, TM), lambda i: (i, 0, 0)),
            pl.BlockSpec((AFEAT, HIDDEN), lambda i: (0, 0)),
            pl.BlockSpec((HIDDEN, HIDDEN), lambda i: (0, 0)),
            pl.BlockSpec((1, HIDDEN), lambda i: (0, 0)),
            pl.BlockSpec((HIDDEN, HIDDEN), lambda i: (0, 0)),
            pl.BlockSpec((QM, HIDDEN), lambda i: (0, 0)),
        ],
        out_specs=pl.BlockSpec((MPB, HIDDEN), lambda i: (i, 0)),
        out_shape=jax.ShapeDtypeStruct((nmol, HIDDEN), jnp.float32),
    )(fa_flat, ga_flat, qm_flat, mask_tiles, Wo_a, Wo_h, bo_row, W0_h, W0_q)


def _attention_head(mol, rxnfp, Wq, bq_row, Wk, bk_row, Wv, bv_row,
                    Wout, bout_row, Wrs_top, Wrs_bot, brs_row):
    def body(m_ref, r_ref, wq_ref, bq_ref, wk_ref, bk_ref, wv_ref, bv_ref,
             wo_ref, bo_ref, wt_ref, wb_ref, br_ref, out_ref):
        m = m_ref[...]
        r = r_ref[...]
        q = jnp.dot(m, wq_ref[...], preferred_element_type=jnp.float32) + bq_ref[...]
        k = jnp.dot(r, wk_ref[...], preferred_element_type=jnp.float32) + bk_ref[...]
        v = jnp.dot(r, wv_ref[...], preferred_element_type=jnp.float32) + bv_ref[...]
        g = (lax.broadcasted_iota(jnp.int32, (HIDDEN, HEADS), 0) // DK
             == lax.broadcasted_iota(jnp.int32, (HIDDEN, HEADS), 1)
             ).astype(jnp.float32)
        gt = (lax.broadcasted_iota(jnp.int32, (HEADS, HIDDEN), 1) // DK
              == lax.broadcasted_iota(jnp.int32, (HEADS, HIDDEN), 0)
              ).astype(jnp.float32)
        s = jnp.dot(q * k, g, preferred_element_type=jnp.float32) \
            * (1.0 / math.sqrt(DK))
        s = s - jnp.max(s, axis=1, keepdims=True)
        e = jnp.exp(s)
        w = e / jnp.sum(e, axis=1, keepdims=True)
        wb = jnp.dot(w, gt, preferred_element_type=jnp.float32)
        att = jnp.dot(wb * v, wo_ref[...],
                      preferred_element_type=jnp.float32) + bo_ref[...]
        out_ref[...] = (
            jnp.dot(m, wt_ref[...], preferred_element_type=jnp.float32)
            + jnp.dot(att, wb_ref[...], preferred_element_type=jnp.float32)
            + br_ref[...])

    return pl.pallas_call(
        body,
        out_shape=jax.ShapeDtypeStruct((B, 1), jnp.float32),
    )(mol, rxnfp, Wq, bq_row, Wk, bk_row, Wv, bv_row, Wout, bout_row,
      Wrs_top, Wrs_bot, brs_row)


def _flat_indices(graph, X):
    nmol = graph.shape[0]
    offs = (jnp.arange(nmol, dtype=jnp.int32) * E)[:, None, None]
    fi = (graph.astype(jnp.int32) + offs)
    fi = jnp.transpose(fi, (0, 2, 1))
    return fi.reshape(NW, (nmol // NW) * MAX_NB * (X // CW), CW)


def kernel(fatoms, fbonds, atom_graph, bond_graph, num_nbs, atom_mask,
           bond_mask, core_mask, fatom_qm, res_rxnfp, Wi, bi, Wh, Wo, bo,
           W_rs0, Wq, bq, Wk, bk, Wv, bv, Wout, bout, W_rs, b_rs):
    bi_row = bi.reshape(1, HIDDEN)
    bo_row = bo.reshape(1, HIDDEN)

    binput, msg = _bond_input(fbonds.reshape(B * E, BFEAT), Wi, bi_row)
    fi_bond = _flat_indices(bond_graph, E)
    fi_atom = _flat_indices(atom_graph, A)

    for _ in range(DEPTH - 1):
        g = _sc_gather_bond(msg, fi_bond)
        msg = _message_update(g, binput, Wh)
    ga = _sc_gather_atom(msg, fi_atom)

    mask_tiles = (atom_mask * core_mask).reshape(B // 8, 1, 8 * A)
    mol = _atom_head(
        fatoms.reshape(B * A, AFEAT), ga, fatom_qm.reshape(B * A, QM),
        mask_tiles, Wo[:AFEAT], Wo[AFEAT:], bo_row,
        W_rs0[:HIDDEN], W_rs0[HIDDEN:])

    return _attention_head(
        mol, res_rxnfp, Wq, bq.reshape(1, HIDDEN), Wk, bk.reshape(1, HIDDEN),
        Wv, bv.reshape(1, HIDDEN), Wout, bout.reshape(1, HIDDEN),
        W_rs[:HIDDEN], W_rs[HIDDEN:], b_rs.reshape(1, 1))
